# Initial kernel scaffold; baseline (speedup 1.0000x reference)
import jax
import jax.numpy as jnp
from jax.experimental import pallas as pl


def kernel(x, edge_index, edge_values, W, b):
    raise NotImplementedError("write your pallas kernel here")


# speedup vs baseline: 4.4222x; 4.4222x over previous
import functools

import jax
import jax.numpy as jnp
from jax import lax
from jax.experimental import pallas as pl
from jax.experimental.pallas import tpu as pltpu
from jax.experimental.pallas import tpu_sc as plsc

N = 10000
E = 320000
F = 128
NC = 2
NS = 16
L = 16
EPC = E // NC
EPT = EPC // NS
K = 80
CHUNKS = EPT // K
RSTEP = 624
RSPAN = 640

_mesh = plsc.VectorSubcoreMesh(core_axis_name="c", subcore_axis_name="s")


@functools.partial(
    pl.kernel,
    out_type=jax.ShapeDtypeStruct((NC, N, F), jnp.float32),
    mesh=_mesh,
    scratch_types=[
        pltpu.VMEM_SHARED((N, F), jnp.float32),
        pltpu.VMEM((K,), jnp.int32),
        pltpu.VMEM((K,), jnp.int32),
        pltpu.VMEM((K,), jnp.float32),
        pltpu.VMEM((K, F), jnp.float32),
        pltpu.SemaphoreType.DMA,
    ],
)
def _sc_aggregate(support, rows, cols, vals, zeros, out,
                  acc, col_v, row_v, val_v, rows_v, sem):
    c = lax.axis_index("c")
    s = lax.axis_index("s")

    rbase = s * RSTEP
    pltpu.sync_copy(zeros.at[pl.ds(rbase, RSPAN)], acc.at[pl.ds(rbase, RSPAN)])
    plsc.subcore_barrier()

    ebase = c * EPC + s * EPT

    def chunk_body(g, _):
        off = ebase + g * K
        pltpu.sync_copy(cols.at[pl.ds(off, K)], col_v)
        pltpu.sync_copy(rows.at[pl.ds(off, K)], row_v)
        pltpu.sync_copy(vals.at[pl.ds(off, K)], val_v)
        pltpu.async_copy(support.at[col_v], rows_v, sem).wait()

        def group_body(t, _):
            vvec = val_v[pl.ds(t * L, L)]
            for e in range(L):
                vb = jnp.full((L,), vvec[e], jnp.float32)
                k = t * L + e
                for j in range(F // L):
                    sl = pl.ds(j * L, L)
                    rows_v[k, sl] = rows_v[k, sl] * vb
            return 0

        lax.fori_loop(0, K // L, group_body, 0)
        pltpu.sync_copy(rows_v, acc.at[row_v], add=True)
        return 0

    lax.fori_loop(0, CHUNKS, chunk_body, 0)
    plsc.subcore_barrier()
    pltpu.sync_copy(acc.at[pl.ds(rbase, RSPAN)], out.at[c, pl.ds(rbase, RSPAN)])


def _mm_body(x_ref, w_ref, o_ref):
    o_ref[...] = jnp.dot(x_ref[...], w_ref[...],
                         preferred_element_type=jnp.float32)


def _combine_body(p_ref, b_ref, o_ref):
    o_ref[...] = p_ref[0] + p_ref[1] + b_ref[...]


_MM_BLK = 1000


def kernel(x, edge_index, edge_values, W, b):
    support = pl.pallas_call(
        _mm_body,
        grid=(N // _MM_BLK,),
        in_specs=[
            pl.BlockSpec((_MM_BLK, F), lambda i: (i, 0)),
            pl.BlockSpec((F, F), lambda i: (0, 0)),
        ],
        out_specs=pl.BlockSpec((_MM_BLK, F), lambda i: (i, 0)),
        out_shape=jax.ShapeDtypeStruct((N, F), jnp.float32),
    )(x, W)

    rows = edge_index[0].astype(jnp.int32)
    cols = edge_index[1].astype(jnp.int32)
    zeros = jnp.zeros((N, F), jnp.float32)
    partials = _sc_aggregate(support, rows, cols,
                             edge_values.astype(jnp.float32), zeros)

    out = pl.pallas_call(
        _combine_body,
        grid=(N // _MM_BLK,),
        in_specs=[
            pl.BlockSpec((NC, _MM_BLK, F), lambda i: (0, i, 0)),
            pl.BlockSpec((1, F), lambda i: (0, 0)),
        ],
        out_specs=pl.BlockSpec((_MM_BLK, F), lambda i: (i, 0)),
        out_shape=jax.ShapeDtypeStruct((N, F), jnp.float32),
    )(partials, b.reshape(1, F))
    return out
